# speedup vs baseline: 4.2278x; 4.2278x over previous
import functools

import jax
import jax.numpy as jnp
from jax import lax
from jax.experimental import pallas as pl
from jax.experimental.pallas import tpu as pltpu
from jax.experimental.pallas import tpu_sc as plsc

VOCAB = 1000000
FACT_DIM = 128
D_MODEL = 1024

_NC = 2
_NS = 16
_NW = _NC * _NS

_IDX_CHUNK = 128


def _gather_body(table_hbm, idx_hbm, out_hbm, idx_v, rows_v, sem, b_per_w):
    wid = lax.axis_index("s") * _NC + lax.axis_index("c")
    base = wid * b_per_w
    pltpu.sync_copy(idx_hbm.at[pl.ds(base, b_per_w)], idx_v)
    n_chunks = b_per_w // _IDX_CHUNK
    copies = []
    for j in range(n_chunks):
        copies.append(
            pltpu.async_copy(
                table_hbm.at[idx_v.at[pl.ds(j * _IDX_CHUNK, _IDX_CHUNK)]],
                rows_v.at[pl.ds(j * _IDX_CHUNK, _IDX_CHUNK)],
                sem,
            )
        )
    for c in copies:
        c.wait()
    pltpu.sync_copy(rows_v, out_hbm.at[pl.ds(base, b_per_w)])


def _sc_gather(table, idx):
    b = idx.shape[0]
    b_per_w = b // _NW
    mesh = plsc.VectorSubcoreMesh(core_axis_name="c", subcore_axis_name="s")
    return pl.kernel(
        functools.partial(_gather_body, b_per_w=b_per_w),
        out_type=jax.ShapeDtypeStruct((b, FACT_DIM), jnp.float32),
        mesh=mesh,
        scratch_types=[
            pltpu.VMEM((b_per_w,), jnp.int32),
            pltpu.VMEM((b_per_w, FACT_DIM), jnp.float32),
            pltpu.SemaphoreType.DMA,
        ],
    )(table, idx)


def _matmul_body(x_ref, w_ref, o_ref):
    o_ref[...] = lax.dot_general(
        x_ref[...],
        w_ref[...],
        (((1,), (1,)), ((), ())),
        preferred_element_type=jnp.float32,
    )


def _tc_project(rows, w):
    b = rows.shape[0]
    blk = 1024
    return pl.pallas_call(
        _matmul_body,
        grid=(b // blk,),
        in_specs=[
            pl.BlockSpec((blk, FACT_DIM), lambda i: (i, 0)),
            pl.BlockSpec((D_MODEL, FACT_DIM), lambda i: (0, 0)),
        ],
        out_specs=pl.BlockSpec((blk, D_MODEL), lambda i: (i, 0)),
        out_shape=jax.ShapeDtypeStruct((b, D_MODEL), jnp.float32),
    )(rows, w)


def kernel(input_ids, token_embedding, projection_weight):
    batch, seq = input_ids.shape
    idx = input_ids.reshape(-1).astype(jnp.int32)
    rows = _sc_gather(token_embedding, idx)
    out = _tc_project(rows, projection_weight)
    return out.reshape(batch, seq, D_MODEL)
